# Initial kernel scaffold; baseline (speedup 1.0000x reference)
import jax
import jax.numpy as jnp
from jax.experimental import pallas as pl


def kernel(users, items, kg_neighbors, all_candidates, neighbor_relations, user_all_embeddings, entity_all_embeddings, item_embeddings, W_ui, b_ui, W1, b1, W2, b2):
    raise NotImplementedError("write your pallas kernel here")


# speedup vs baseline: 12.2074x; 12.2074x over previous
import functools

import jax
import jax.numpy as jnp
from jax import lax
from jax.experimental import pallas as pl
from jax.experimental.pallas import tpu as pltpu
from jax.experimental.pallas import tpu_sc as plsc

_B = 1024
_D = 64
_K = 32
_NC = 64
_RC = 16
_R = 4
_NREL = 50
_NI = 100000
_CF = _NREL * _NC
_CP = 3328

_F32 = jnp.float32
_I32 = jnp.int32
_HI = lax.Precision.HIGHEST


def _sc_mesh():
    return plsc.VectorSubcoreMesh(core_axis_name="c", subcore_axis_name="s")


def _sc_params():
    return pltpu.CompilerParams(use_tc_tiling_on_sc=False)


def _sc_gather1(items, users, kg, nrel_tab, uemb, eemb, acand_flat):
    out_type = (
        jax.ShapeDtypeStruct((_B, _K), _I32),
        jax.ShapeDtypeStruct((_B, _K), _I32),
        jax.ShapeDtypeStruct((_B, _D), _F32),
        jax.ShapeDtypeStruct((_CP, _D), _F32),
    )

    @functools.partial(
        pl.kernel, out_type=out_type, mesh=_sc_mesh(),
        compiler_params=_sc_params(),
        scratch_types=[
            pltpu.VMEM((32,), _I32),
            pltpu.VMEM((32,), _I32),
            pltpu.VMEM((104,), _I32),
            pltpu.VMEM((32, _K), _I32),
            pltpu.VMEM((32, _K), _I32),
            pltpu.VMEM((32, _D), _F32),
            pltpu.VMEM((104, _D), _F32),
            pltpu.SemaphoreType.DMA,
        ])
    def k(items_h, users_h, kg_h, nrl_h, ue_h, ee_h, ac_h,
          nbr_o, nrl_o, uemb_o, ce_o,
          it_v, us_v, ci_v, nbr_v, nrl_v, ue_v, ce_v, sem):
        w = lax.axis_index("c") * 16 + lax.axis_index("s")
        b0 = w * 32
        c0 = w * 104
        pltpu.sync_copy(items_h.at[pl.ds(b0, 32)], it_v)
        pltpu.sync_copy(users_h.at[pl.ds(b0, 32)], us_v)
        pltpu.sync_copy(ac_h.at[pl.ds(c0, 104)], ci_v)
        d1 = pltpu.async_copy(kg_h.at[it_v], nbr_v, sem)
        d2 = pltpu.async_copy(nrl_h.at[it_v], nrl_v, sem)
        d3 = pltpu.async_copy(ue_h.at[us_v], ue_v, sem)
        d4 = pltpu.async_copy(ee_h.at[ci_v], ce_v, sem)
        d1.wait()
        d2.wait()
        d3.wait()
        d4.wait()
        pltpu.sync_copy(nbr_v, nbr_o.at[pl.ds(b0, 32)])
        pltpu.sync_copy(nrl_v, nrl_o.at[pl.ds(b0, 32)])
        pltpu.sync_copy(ue_v, uemb_o.at[pl.ds(b0, 32)])
        pltpu.sync_copy(ce_v, ce_o.at[pl.ds(c0, 104)])

    return k(items, users, kg, nrel_tab, uemb, eemb, acand_flat)


def _sc_gather2(nbr_flat, eemb):
    out_type = jax.ShapeDtypeStruct((_B * _K, _D), _F32)

    @functools.partial(
        pl.kernel, out_type=out_type, mesh=_sc_mesh(),
        compiler_params=_sc_params(),
        scratch_types=[
            pltpu.VMEM((1024,), _I32),
            pltpu.VMEM((1024, _D), _F32),
            pltpu.SemaphoreType.DMA,
        ])
    def k(idx_h, ee_h, out_o, idx_v, rows_v, sem):
        w = lax.axis_index("c") * 16 + lax.axis_index("s")
        base = w * 1024
        pltpu.sync_copy(idx_h.at[pl.ds(base, 1024)], idx_v)
        copies = []
        for j in range(8):
            copies.append(pltpu.async_copy(
                ee_h.at[idx_v.at[pl.ds(j * 128, 128)]],
                rows_v.at[pl.ds(j * 128, 128)], sem))
        for c in copies:
            c.wait()
        pltpu.sync_copy(rows_v, out_o.at[pl.ds(base, 1024)])

    return k(nbr_flat, eemb)


_GB = 8
_BB = _B // _GB


def _b1_body(ue_r, ie_r, wui_r, bui_r, nr_r, w1_r, b1_r, nrl_r,
             ui_o, ap1_o, idx1_o, rel4_o, repl_o):
    ue = ue_r[...]
    ie = ie_r[...]
    n = ue.shape[0]
    wui = wui_r[...]
    ui = (jnp.dot(ue, wui[:, :_D].T, precision=_HI, preferred_element_type=_F32)
          + jnp.dot(ie, wui[:, _D:].T, precision=_HI, preferred_element_type=_F32)
          + bui_r[...])
    ui = ui / jnp.maximum(jnp.sqrt(jnp.sum(ui * ui, axis=1, keepdims=True)), 1e-12)
    ne = (jnp.dot(nr_r[...], w1_r[...].T, precision=_HI, preferred_element_type=_F32)
          + b1_r[...])
    ne = ne / jnp.maximum(jnp.sqrt(jnp.sum(ne * ne, axis=1, keepdims=True)), 1e-12)
    ne3 = ne.reshape(n, _K, _D)
    s1 = jnp.sum(ui[:, None, :] * ne3, axis=2)
    m = jnp.max(s1, axis=1, keepdims=True)
    e = jnp.exp(s1 - m)
    p1 = e / jnp.sum(e, axis=1, keepdims=True)
    iota = lax.broadcasted_iota(_I32, (n, _K), 1)
    nrl = nrl_r[...]
    x = p1
    idxs, rels, probs, repls = [], [], [], []
    for _ in range(_R):
        idx = jnp.argmax(x, axis=1)[:, None].astype(_I32)
        oh = iota == idx
        probs.append(jnp.sum(jnp.where(oh, p1, 0.0), axis=1, keepdims=True))
        rels.append(jnp.sum(jnp.where(oh, nrl, 0), axis=1, keepdims=True))
        repls.append(jnp.sum(ne3 * oh.astype(_F32)[:, :, None], axis=1))
        idxs.append(idx)
        x = jnp.where(oh, -1.0, x)
    ap1 = jnp.log((probs[0] + probs[1] + probs[2] + probs[3]) * 0.25)
    ui_o[...] = ui
    ap1_o[...] = ap1
    idx1_o[...] = jnp.concatenate(idxs, axis=1)
    rel4_o[...] = jnp.concatenate(rels, axis=1)
    repl_o[...] = jnp.concatenate(repls, axis=1)


def _tc_step1(users_e, item_e, wui, bui, nbr_raw, w1, b1, nrel_g):
    out_shape = (
        jax.ShapeDtypeStruct((_B, _D), _F32),
        jax.ShapeDtypeStruct((_B, 1), _F32),
        jax.ShapeDtypeStruct((_B, _R), _I32),
        jax.ShapeDtypeStruct((_B, _R), _I32),
        jax.ShapeDtypeStruct((_B, _R * _D), _F32),
    )
    full = lambda shape: pl.BlockSpec(shape, lambda i: (0, 0))
    blk = lambda shape: pl.BlockSpec(shape, lambda i: (i, 0))
    return pl.pallas_call(
        _b1_body,
        grid=(_GB,),
        in_specs=[
            blk((_BB, _D)), blk((_BB, _D)), full((_D, 2 * _D)), full((1, _D)),
            blk((_BB * _K, _D)), full((_D, _D)), full((1, _D)), blk((_BB, _K)),
        ],
        out_specs=(
            blk((_BB, _D)), blk((_BB, 1)), blk((_BB, _R)), blk((_BB, _R)),
            blk((_BB, _R * _D)),
        ),
        out_shape=out_shape)(
        users_e, item_e, wui, bui, nbr_raw, w1, b1, nrel_g)


def _b2_body(ui_r, repl_r, rel4_r, ce0_r, ue_r, w2_r, b2_r, ac_r,
             gum_r, ap1_r,
             ap2_o, sel4_o):
    ui = ui_r[...]
    ue = ue_r[...]
    n = ui.shape[0]
    ce0 = ce0_r[...]
    ce2 = (jnp.dot(ce0, w2_r[...].T, precision=_HI, preferred_element_type=_F32)
           + b2_r[...])
    ce2 = ce2 / jnp.maximum(jnp.sqrt(jnp.sum(ce2 * ce2, axis=1, keepdims=True)), 1e-12)
    sc_all = jnp.dot(ue, ce0.T, precision=_HI, preferred_element_type=_F32)
    acf = ac_r[...].astype(_F32)
    iota50 = lax.broadcasted_iota(_I32, (n, _NREL), 1)
    iota64 = lax.broadcasted_iota(_I32, (n, _NC), 1)
    iota16 = lax.broadcasted_iota(_I32, (n, _RC), 1)
    rel4 = rel4_r[...]
    gum = gum_r[...]
    repl = repl_r[...]
    qs = [ui * repl[:, r * _D:(r + 1) * _D] for r in range(_R)]
    s2f = jnp.dot(jnp.concatenate(qs, axis=0), ce2.T,
                  precision=_HI, preferred_element_type=_F32)
    p2s, sels = [], []
    for r in range(_R):
        relf = rel4[:, r:r + 1]
        ohrel = (iota50 == relf).astype(_F32)
        crow = jnp.dot(ohrel, acf, precision=_HI, preferred_element_type=_F32)
        s2fr = s2f[r * n:(r + 1) * n]
        srow = jnp.zeros((n, _NC), _F32)
        s2row = jnp.zeros((n, _NC), _F32)
        for p in range(_NREL):
            m = (relf == p).astype(_F32)
            srow = srow + m * sc_all[:, p * _NC:(p + 1) * _NC]
            s2row = s2row + m * s2fr[:, p * _NC:(p + 1) * _NC]
        x = srow
        s2cols, ccols = [], []
        for _ in range(_RC):
            idx = jnp.argmax(x, axis=1)[:, None].astype(_I32)
            oh = iota64 == idx
            s2cols.append(jnp.sum(jnp.where(oh, s2row, 0.0), axis=1, keepdims=True))
            ccols.append(jnp.sum(jnp.where(oh, crow, 0.0), axis=1, keepdims=True))
            x = jnp.where(oh, -1e30, x)
        s2sel = jnp.concatenate(s2cols, axis=1)
        csel = jnp.concatenate(ccols, axis=1)
        mx = jnp.max(s2sel, axis=1, keepdims=True)
        e = jnp.exp(s2sel - mx)
        l2 = e / jnp.sum(e, axis=1, keepdims=True)
        y = jnp.log(l2 + 1e-20) + gum[:, r * _RC:(r + 1) * _RC]
        idx2 = jnp.argmax(y, axis=1)[:, None].astype(_I32)
        oh2 = iota16 == idx2
        p2s.append(jnp.sum(jnp.where(oh2, l2, 0.0), axis=1, keepdims=True))
        sels.append(jnp.sum(jnp.where(oh2, csel, 0.0), axis=1, keepdims=True))
    ap2_o[...] = ap1_r[...] + jnp.log((p2s[0] + p2s[1] + p2s[2] + p2s[3]) * 0.25)
    sel4_o[...] = jnp.concatenate(sels, axis=1)


def _tc_step2(ui, repl, rel4, ce0, users_e, w2, b2, acand, gum, ap1):
    out_shape = (
        jax.ShapeDtypeStruct((_B, 1), _F32),
        jax.ShapeDtypeStruct((_B, _R), _F32),
    )
    full = lambda shape: pl.BlockSpec(shape, lambda i: (0, 0))
    blk = lambda shape: pl.BlockSpec(shape, lambda i: (i, 0))
    return pl.pallas_call(
        _b2_body,
        grid=(_GB,),
        in_specs=[
            blk((_BB, _D)), blk((_BB, _R * _D)), blk((_BB, _R)),
            full((_CF, _D)), blk((_BB, _D)), full((_D, _D)), full((1, _D)),
            full((_NREL, _NC)), blk((_BB, _R * _RC)), blk((_BB, 1)),
        ],
        out_specs=(blk((_BB, 1)), blk((_BB, _R))),
        out_shape=out_shape)(
        ui, repl, rel4, ce0, users_e, w2, b2, acand, gum, ap1)


def _b3_body(itc_r, itr_r, idx1_r, sel4_r, nbr_r, row1_o, row2_o):
    itc = itc_r[...]
    itr = itr_r[...]
    mm = itc == itr
    bc = lax.broadcasted_iota(_I32, (_B, _B), 0)
    br = lax.broadcasted_iota(_I32, (_B, _B), 1)
    mup = jnp.where(mm & (br > bc), 1.0, 0.0).astype(jnp.bfloat16)
    mb = jnp.where(mm, 1.0, 0.0).astype(jnp.bfloat16)
    idx1 = idx1_r[...]
    sel4 = sel4_r[...]
    iota32 = lax.broadcasted_iota(_I32, (_B, _K), 1)
    ohs1 = [iota32 == idx1[:, r:r + 1] for r in range(_R)]
    u1 = ohs1[0] | ohs1[1] | ohs1[2] | ohs1[3]
    vals = jnp.zeros((_B, _K), _F32)
    for r in range(_R):
        vals = jnp.where(ohs1[r], sel4[:, r:r + 1], vals)
    u1f = u1.astype(jnp.bfloat16)
    later = jnp.dot(mup, u1f, preferred_element_type=_F32)
    surv = u1 & (later == 0.0)
    sval = jnp.where(surv, vals, 0.0)
    c2 = jnp.floor(sval / 65536.0)
    rem = sval - c2 * 65536.0
    c1 = jnp.floor(rem / 256.0)
    c0 = rem - c1 * 256.0
    vals2 = (jnp.dot(mb, c0.astype(jnp.bfloat16), preferred_element_type=_F32)
             + 256.0 * jnp.dot(mb, c1.astype(jnp.bfloat16), preferred_element_type=_F32)
             + 65536.0 * jnp.dot(mb, c2.astype(jnp.bfloat16), preferred_element_type=_F32))
    union = jnp.dot(mb, u1f, preferred_element_type=_F32) > 0.5
    kgrow = nbr_r[...]
    row1_o[...] = jnp.where(union, 0, kgrow)
    row2_o[...] = jnp.where(union, vals2.astype(_I32), kgrow)


def _tc_merge(items_c, items_r, idx1, sel4, neighbors):
    out_shape = (
        jax.ShapeDtypeStruct((_B, _K), _I32),
        jax.ShapeDtypeStruct((_B, _K), _I32),
    )
    return pl.pallas_call(_b3_body, out_shape=out_shape)(
        items_c, items_r, idx1, sel4, neighbors)


def _sc_scatter(kg, items2d, rows1, rows2):
    out_type = (
        jax.ShapeDtypeStruct((_NI, _K), _I32),
        jax.ShapeDtypeStruct((_NI, _K), _I32),
    )
    nch = 10
    ch = 625

    @functools.partial(
        pl.kernel, out_type=out_type, mesh=_sc_mesh(),
        compiler_params=_sc_params(),
        scratch_types=[
            pltpu.VMEM((1, 64), _I32),
            pltpu.VMEM((64, _K), _I32),
            pltpu.VMEM((ch, _K), _I32),
            pltpu.VMEM((ch, _K), _I32),
            pltpu.SemaphoreType.DMA,
            pltpu.SemaphoreType.DMA,
            pltpu.SemaphoreType.DMA,
        ])
    def k(kg_h, it_h, r1_h, r2_h, cf1_o, cf2_o, idx_v, rv, buf0, buf1,
          rsem, wsem, sem):
        c = lax.axis_index("c")
        s = lax.axis_index("s")

        def copy_all(dst):
            base = s * (nch * ch)
            bufs = (buf0, buf1)
            reads = [None] * nch
            writes = [None] * nch
            reads[0] = pltpu.async_copy(
                kg_h.at[pl.ds(base, ch)], bufs[0], rsem)
            for i in range(nch):
                if i + 1 < nch:
                    if i >= 1:
                        writes[i - 1].wait()
                    reads[i + 1] = pltpu.async_copy(
                        kg_h.at[pl.ds(base + (i + 1) * ch, ch)],
                        bufs[(i + 1) % 2], rsem)
                reads[i].wait()
                writes[i] = pltpu.async_copy(
                    bufs[i % 2], dst.at[pl.ds(base + i * ch, ch)], wsem)
            writes[nch - 2].wait()
            writes[nch - 1].wait()

        @pl.when(c == 0)
        def _():
            copy_all(cf1_o)

        @pl.when(c == 1)
        def _():
            copy_all(cf2_o)

        plsc.subcore_barrier()
        pltpu.sync_copy(it_h.at[pl.ds(s, 1)], idx_v)

        @pl.when(c == 0)
        def _():
            pltpu.sync_copy(r1_h.at[pl.ds(s * 64, 64)], rv)
            pltpu.sync_copy(rv, cf1_o.at[idx_v.at[0]])

        @pl.when(c == 1)
        def _():
            pltpu.sync_copy(r2_h.at[pl.ds(s * 64, 64)], rv)
            pltpu.sync_copy(rv, cf2_o.at[idx_v.at[0]])

    return k(kg, items2d, rows1, rows2)


def kernel(users, items, kg_neighbors, all_candidates, neighbor_relations,
           user_all_embeddings, entity_all_embeddings, item_embeddings,
           W_ui, b_ui, W1, b1, W2, b2):
    users = users.astype(_I32)
    items = items.astype(_I32)
    kg_neighbors = kg_neighbors.astype(_I32)
    all_candidates = all_candidates.astype(_I32)
    neighbor_relations = neighbor_relations.astype(_I32)
    acand_flat = jnp.pad(all_candidates.reshape(-1), (0, _CP - _CF))
    neighbors, nrelg, users_e, ce0p = _sc_gather1(
        items, users, kg_neighbors, neighbor_relations,
        user_all_embeddings, entity_all_embeddings, acand_flat)
    nbre = _sc_gather2(neighbors.reshape(-1), entity_all_embeddings)
    ui, ap1, idx1, rel4, repl = _tc_step1(
        users_e, item_embeddings, W_ui, b_ui.reshape(1, _D), nbre,
        W1, b1.reshape(1, _D), nrelg)
    gum = jax.random.gumbel(jax.random.fold_in(jax.random.key(0), 7),
                            (_B * _R, _RC), _F32).reshape(_B, _R * _RC)
    ap2, sel4 = _tc_step2(
        ui, repl, rel4, ce0p[:_CF], users_e, W2, b2.reshape(1, _D),
        all_candidates, gum, ap1)
    row1, row2 = _tc_merge(items.reshape(_B, 1), items.reshape(1, _B),
                           idx1, sel4, neighbors)
    cf1, cf2 = _sc_scatter(kg_neighbors, items.reshape(16, 64), row1, row2)
    return (cf1, ap1.reshape(_B), cf2, ap2.reshape(_B))
